# baseline (speedup 1.0000x reference)
import math

import jax
import jax.numpy as jnp
from jax import lax
from jax.experimental import pallas as pl
from jax.experimental.pallas import tpu as pltpu
from jax.experimental.pallas import tpu_sc as plsc

N_NODES = 10000
N_EDGES = 160000
MUL = 32
WN = MUL * MUL
PATH_W = 1.0 / math.sqrt(MUL)
EXT = 128
GROUP = 128
NWORKERS = 32
TRIPS = 40
E_PAD = NWORKERS * TRIPS * GROUP
N_PAD = 10240
ROWS_PER_TILE = N_PAD // 16

EDGE_BLK = 2000
NODE_BLK = 1000


def _edge_body(ea_ref, xs_ref, sh_ref, w1_ref, b1_ref, w2_ref, b2_ref, out_ref):
    ea = ea_ref[...]
    h = jnp.maximum(
        jnp.dot(ea, w1_ref[...], preferred_element_type=jnp.float32) + b1_ref[...],
        0.0)
    w = jnp.dot(h.astype(jnp.bfloat16), w2_ref[...],
                preferred_element_type=jnp.float32) + b2_ref[...]
    xs2 = (xs_ref[...] * (PATH_W * sh_ref[...])).astype(jnp.bfloat16)
    jcol = lax.broadcasted_iota(jnp.int32, (MUL, WN), 1)
    urow = lax.broadcasted_iota(jnp.int32, (MUL, WN), 0)
    tmat = (jcol % MUL == urow).astype(jnp.bfloat16)
    x_exp = jnp.dot(xs2, tmat, preferred_element_type=jnp.float32)
    jrow = lax.broadcasted_iota(jnp.int32, (WN, MUL), 0)
    wcol = lax.broadcasted_iota(jnp.int32, (WN, MUL), 1)
    smat = (jrow // MUL == wcol).astype(jnp.bfloat16)
    msg = jnp.dot((x_exp * w).astype(jnp.bfloat16), smat,
                  preferred_element_type=jnp.float32)
    ones = jnp.ones((msg.shape[0], EXT - MUL), jnp.float32)
    out_ref[...] = jnp.concatenate([msg, ones], axis=1)


def _edge_stage(edge_attr, x_src, sh, w1, b1, w2, b2):
    nblk = N_EDGES // EDGE_BLK
    return pl.pallas_call(
        _edge_body,
        grid=(nblk,),
        in_specs=[
            pl.BlockSpec((EDGE_BLK, 16), lambda i: (i, 0)),
            pl.BlockSpec((EDGE_BLK, MUL), lambda i: (i, 0)),
            pl.BlockSpec((EDGE_BLK, 1), lambda i: (i, 0)),
            pl.BlockSpec((16, 64), lambda i: (0, 0)),
            pl.BlockSpec((1, 64), lambda i: (0, 0)),
            pl.BlockSpec((64, WN), lambda i: (0, 0)),
            pl.BlockSpec((1, WN), lambda i: (0, 0)),
        ],
        out_specs=pl.BlockSpec((EDGE_BLK, EXT), lambda i: (i, 0)),
        out_shape=jax.ShapeDtypeStruct((E_PAD, EXT), jnp.float32),
    )(edge_attr, x_src, sh, w1, b1.reshape(1, 64),
      w2.reshape(64, MUL, MUL).transpose(0, 2, 1).reshape(64, WN)
      .astype(jnp.bfloat16),
      b2.reshape(MUL, MUL).T.reshape(1, WN))


def _scatter_body(msg_hbm, dst_hbm, zeros_hbm, acc_hbm, idx_v, msg_v, acc_sh):
    c = lax.axis_index("c")
    s = lax.axis_index("s")
    wid = c * 16 + s
    row0 = s * ROWS_PER_TILE
    pltpu.sync_copy(zeros_hbm.at[pl.ds(row0, ROWS_PER_TILE)],
                    acc_sh.at[pl.ds(row0, ROWS_PER_TILE)])
    plsc.subcore_barrier()

    g0 = wid * TRIPS

    def body(i, carry):
        pltpu.sync_copy(dst_hbm.at[wid, i], idx_v)
        pltpu.sync_copy(msg_hbm.at[g0 + i], msg_v)
        pltpu.sync_copy(msg_v, acc_sh.at[idx_v], add=True)
        return carry

    lax.fori_loop(0, TRIPS, body, 0)
    plsc.subcore_barrier()
    pltpu.sync_copy(acc_sh.at[pl.ds(row0, ROWS_PER_TILE)],
                    acc_hbm.at[c, pl.ds(row0, ROWS_PER_TILE)])


def _scatter_stage(msgext, dst_pad):
    mesh = plsc.VectorSubcoreMesh(core_axis_name="c", subcore_axis_name="s")
    zeros = jnp.zeros((N_PAD, EXT), jnp.float32)
    run = pl.kernel(
        _scatter_body,
        out_type=jax.ShapeDtypeStruct((2, N_PAD, EXT), jnp.float32),
        mesh=mesh,
        scratch_types=[
            pltpu.VMEM((GROUP,), jnp.int32),
            pltpu.VMEM((GROUP, EXT), jnp.float32),
            pltpu.VMEM_SHARED((N_PAD, EXT), jnp.float32),
        ],
    )
    return run(msgext.reshape(NWORKERS * TRIPS, GROUP, EXT),
               dst_pad.reshape(NWORKERS, TRIPS, GROUP), zeros)


def _final_body(acc0_ref, acc1_ref, xd_ref, wres_ref, out_ref):
    tot = acc0_ref[...] + acc1_ref[...]
    summed = tot[:, :MUL]
    cnt = tot[:, MUL:MUL + 1]
    agg = summed / jnp.maximum(cnt, 1.0)
    res = jnp.dot(xd_ref[...], wres_ref[...],
                  preferred_element_type=jnp.float32) * (1.0 / math.sqrt(MUL))
    out_ref[...] = jnp.maximum(res + agg, 0.0)


def _final_stage(acc, x_dst, wres):
    nblk = N_NODES // NODE_BLK
    return pl.pallas_call(
        _final_body,
        grid=(nblk,),
        in_specs=[
            pl.BlockSpec((NODE_BLK, EXT), lambda i: (i, 0)),
            pl.BlockSpec((NODE_BLK, EXT), lambda i: (i, 0)),
            pl.BlockSpec((NODE_BLK, MUL), lambda i: (i, 0)),
            pl.BlockSpec((MUL, MUL), lambda i: (0, 0)),
        ],
        out_specs=pl.BlockSpec((NODE_BLK, MUL), lambda i: (i, 0)),
        out_shape=jax.ShapeDtypeStruct((N_NODES, MUL), jnp.float32),
    )(acc[0], acc[1], x_dst, wres)


def _pad_rows(a, n):
    return jnp.pad(a, ((0, n - a.shape[0]),) + ((0, 0),) * (a.ndim - 1))


def kernel(dst, x_src, x_dst, sh, edge_attr, W1, b1, W2, b2, Wres):
    dst_p = jnp.concatenate(
        [dst, jnp.full((E_PAD - N_EDGES,), N_NODES, jnp.int32)])
    msgext = _edge_stage(edge_attr, x_src, sh, W1, b1, W2, b2)
    acc = _scatter_stage(msgext, dst_p)
    return _final_stage(acc, x_dst, Wres)


# speedup vs baseline: 1.1858x; 1.1858x over previous
import math

import jax
import jax.numpy as jnp
from jax import lax
from jax.experimental import pallas as pl
from jax.experimental.pallas import tpu as pltpu
from jax.experimental.pallas import tpu_sc as plsc

N_NODES = 10000
N_EDGES = 160000
MUL = 32
WN = MUL * MUL
PATH_W = 1.0 / math.sqrt(MUL)
EXT = 128
GROUP = 128
NWORKERS = 32
TRIPS = 40
E_PAD = NWORKERS * TRIPS * GROUP
N_PAD = 10240
ROWS_PER_TILE = N_PAD // 16

EDGE_BLK = 2000
NODE_BLK = 1000


def _edge_body(ea_ref, xs_ref, sh_ref, w1_ref, b1_ref, w2_ref, b2_ref, out_ref):
    ea = ea_ref[...]
    h = jnp.maximum(
        jnp.dot(ea, w1_ref[...], preferred_element_type=jnp.float32) + b1_ref[...],
        0.0)
    w = jnp.dot(h.astype(jnp.bfloat16), w2_ref[...],
                preferred_element_type=jnp.float32) + b2_ref[...]
    xs2 = xs_ref[...] * (PATH_W * sh_ref[...])
    x_exp = jnp.concatenate([xs2] * MUL, axis=1)
    jrow = lax.broadcasted_iota(jnp.int32, (WN, MUL), 0)
    wcol = lax.broadcasted_iota(jnp.int32, (WN, MUL), 1)
    smat = (jrow // MUL == wcol).astype(jnp.bfloat16)
    msg = jnp.dot((x_exp * w).astype(jnp.bfloat16), smat,
                  preferred_element_type=jnp.float32)
    ones = jnp.ones((msg.shape[0], EXT - MUL), jnp.float32)
    out_ref[...] = jnp.concatenate([msg, ones], axis=1)


def _edge_stage(edge_attr, x_src, sh, w1, b1, w2, b2):
    nblk = N_EDGES // EDGE_BLK
    return pl.pallas_call(
        _edge_body,
        grid=(nblk,),
        in_specs=[
            pl.BlockSpec((EDGE_BLK, 16), lambda i: (i, 0)),
            pl.BlockSpec((EDGE_BLK, MUL), lambda i: (i, 0)),
            pl.BlockSpec((EDGE_BLK, 1), lambda i: (i, 0)),
            pl.BlockSpec((16, 64), lambda i: (0, 0)),
            pl.BlockSpec((1, 64), lambda i: (0, 0)),
            pl.BlockSpec((64, WN), lambda i: (0, 0)),
            pl.BlockSpec((1, WN), lambda i: (0, 0)),
        ],
        out_specs=pl.BlockSpec((EDGE_BLK, EXT), lambda i: (i, 0)),
        out_shape=jax.ShapeDtypeStruct((E_PAD, EXT), jnp.float32),
    )(edge_attr, x_src, sh, w1, b1.reshape(1, 64),
      w2.reshape(64, MUL, MUL).transpose(0, 2, 1).reshape(64, WN)
      .astype(jnp.bfloat16),
      b2.reshape(MUL, MUL).T.reshape(1, WN))


def _scatter_body(msg_hbm, dst_hbm, zeros_hbm, acc_hbm, idx_v, msg_v, acc_sh):
    c = lax.axis_index("c")
    s = lax.axis_index("s")
    wid = c * 16 + s
    row0 = s * ROWS_PER_TILE
    pltpu.sync_copy(zeros_hbm.at[pl.ds(row0, ROWS_PER_TILE)],
                    acc_sh.at[pl.ds(row0, ROWS_PER_TILE)])
    plsc.subcore_barrier()

    g0 = wid * TRIPS

    def body(i, carry):
        pltpu.sync_copy(dst_hbm.at[wid, i], idx_v)
        pltpu.sync_copy(msg_hbm.at[g0 + i], msg_v)
        pltpu.sync_copy(msg_v, acc_sh.at[idx_v], add=True)
        return carry

    lax.fori_loop(0, TRIPS, body, 0)
    plsc.subcore_barrier()
    pltpu.sync_copy(acc_sh.at[pl.ds(row0, ROWS_PER_TILE)],
                    acc_hbm.at[c, pl.ds(row0, ROWS_PER_TILE)])


def _scatter_stage(msgext, dst_pad):
    mesh = plsc.VectorSubcoreMesh(core_axis_name="c", subcore_axis_name="s")
    zeros = jnp.zeros((N_PAD, EXT), jnp.float32)
    run = pl.kernel(
        _scatter_body,
        out_type=jax.ShapeDtypeStruct((2, N_PAD, EXT), jnp.float32),
        mesh=mesh,
        scratch_types=[
            pltpu.VMEM((GROUP,), jnp.int32),
            pltpu.VMEM((GROUP, EXT), jnp.float32),
            pltpu.VMEM_SHARED((N_PAD, EXT), jnp.float32),
        ],
    )
    return run(msgext.reshape(NWORKERS * TRIPS, GROUP, EXT),
               dst_pad.reshape(NWORKERS, TRIPS, GROUP), zeros)


def _final_body(acc0_ref, acc1_ref, xd_ref, wres_ref, out_ref):
    tot = acc0_ref[...] + acc1_ref[...]
    summed = tot[:, :MUL]
    cnt = tot[:, MUL:MUL + 1]
    agg = summed / jnp.maximum(cnt, 1.0)
    res = jnp.dot(xd_ref[...], wres_ref[...],
                  preferred_element_type=jnp.float32) * (1.0 / math.sqrt(MUL))
    out_ref[...] = jnp.maximum(res + agg, 0.0)


def _final_stage(acc, x_dst, wres):
    nblk = N_NODES // NODE_BLK
    return pl.pallas_call(
        _final_body,
        grid=(nblk,),
        in_specs=[
            pl.BlockSpec((NODE_BLK, EXT), lambda i: (i, 0)),
            pl.BlockSpec((NODE_BLK, EXT), lambda i: (i, 0)),
            pl.BlockSpec((NODE_BLK, MUL), lambda i: (i, 0)),
            pl.BlockSpec((MUL, MUL), lambda i: (0, 0)),
        ],
        out_specs=pl.BlockSpec((NODE_BLK, MUL), lambda i: (i, 0)),
        out_shape=jax.ShapeDtypeStruct((N_NODES, MUL), jnp.float32),
    )(acc[0], acc[1], x_dst, wres)


def _pad_rows(a, n):
    return jnp.pad(a, ((0, n - a.shape[0]),) + ((0, 0),) * (a.ndim - 1))


def kernel(dst, x_src, x_dst, sh, edge_attr, W1, b1, W2, b2, Wres):
    dst_p = jnp.concatenate(
        [dst, jnp.full((E_PAD - N_EDGES,), N_NODES, jnp.int32)])
    msgext = _edge_stage(edge_attr, x_src, sh, W1, b1, W2, b2)
    acc = _scatter_stage(msgext, dst_p)
    return _final_stage(acc, x_dst, Wres)
